# Initial kernel scaffold; baseline (speedup 1.0000x reference)
import jax
import jax.numpy as jnp
from jax.experimental import pallas as pl


def kernel(edge_index, values, x):
    raise NotImplementedError("write your pallas kernel here")


# speedup vs baseline: 2.5647x; 2.5647x over previous
import functools

import jax
import jax.numpy as jnp
from jax import lax
from jax.experimental import pallas as pl
from jax.experimental.pallas import tpu as pltpu
from jax.experimental.pallas import tpu_sc as plsc

N = 10000
E = 160000
D = 256
DH = 128
NC = 2
NS = 16
L = 16

EDGES_PER_SUB = E // NS
K = 80
NBATCH = EDGES_PER_SUB // K
ROWS_PER_SUB = N // NS
STRIPE = 125


def _gcn_body(row_hbm, col_hbm, val_hbm, x2_hbm, out_hbm,
              row_v, idx_v, val_v, rows_v, stage_v, acc_sh, sem):
    c = lax.axis_index("c")
    s = lax.axis_index("s")

    def _zrow(i, carry):
        for j in range(DH // L):
            stage_v[i, pl.ds(j * L, L)] = jnp.zeros((L,), jnp.float32)
        return carry
    lax.fori_loop(0, STRIPE, _zrow, 0)
    for t in range(ROWS_PER_SUB // STRIPE):
        pltpu.sync_copy(stage_v, acc_sh.at[pl.ds(s * ROWS_PER_SUB + t * STRIPE, STRIPE)])
    plsc.subcore_barrier()

    ebase = s * EDGES_PER_SUB

    def _batch(b, carry):
        base = pl.multiple_of(ebase + b * K, 8)
        pltpu.sync_copy(row_hbm.at[pl.ds(base, K)], row_v)
        pltpu.sync_copy(col_hbm.at[pl.ds(base, K)], idx_v)
        pltpu.sync_copy(val_hbm.at[pl.ds(base, K)], val_v)
        for j in range(K // L):
            v = idx_v[pl.ds(j * L, L)]
            idx_v[pl.ds(j * L, L)] = v * 2 + c
        pltpu.async_copy(x2_hbm.at[idx_v], rows_v, sem).wait()

        def _scale(e, carry2):
            vv = plsc.load_gather(val_v, [jnp.full((L,), e, jnp.int32)])
            for j in range(DH // L):
                rows_v[e, pl.ds(j * L, L)] = rows_v[e, pl.ds(j * L, L)] * vv
            return carry2
        lax.fori_loop(0, K, _scale, 0)

        pltpu.sync_copy(rows_v, acc_sh.at[row_v], add=True)
        return carry
    lax.fori_loop(0, NBATCH, _batch, 0)
    plsc.subcore_barrier()

    for t in range(ROWS_PER_SUB // STRIPE):
        r0 = s * ROWS_PER_SUB + t * STRIPE
        pltpu.sync_copy(acc_sh.at[pl.ds(r0, STRIPE)], stage_v)
        pltpu.sync_copy(stage_v, out_hbm.at[pl.ds(r0, STRIPE), c])


_gcn = pl.kernel(
    _gcn_body,
    out_type=jax.ShapeDtypeStruct((N, NC, DH), jnp.float32),
    mesh=plsc.VectorSubcoreMesh(core_axis_name="c", subcore_axis_name="s"),
    compiler_params=pltpu.CompilerParams(needs_layout_passes=False),
    scratch_types=[
        pltpu.VMEM((K,), jnp.int32),
        pltpu.VMEM((K,), jnp.int32),
        pltpu.VMEM((K,), jnp.float32),
        pltpu.VMEM((K, DH), jnp.float32),
        pltpu.VMEM((STRIPE, DH), jnp.float32),
        pltpu.VMEM_SHARED((N, DH), jnp.float32),
        pltpu.SemaphoreType.DMA,
    ],
)


def kernel(edge_index, values, x):
    row = edge_index[0]
    col = edge_index[1]
    x2 = x.reshape(2 * N, DH)
    out3 = _gcn(row, col, values, x2)
    return out3.reshape(N, D)


# speedup vs baseline: 4.2384x; 1.6526x over previous
import jax
import jax.numpy as jnp
from jax import lax
from jax.experimental import pallas as pl
from jax.experimental.pallas import tpu as pltpu
from jax.experimental.pallas import tpu_sc as plsc

N = 10000
E = 160000
D = 256
DH = 64
NQ = 4
NC = 2
NS = 16
L = 16

EPS = E // NS
K = 80
NBATCH = EPS // K
ROWS_PER_SUB = N // NS
STRIPE = 125


def _gcn_body(row3, colv, valv, x4, out_hbm,
              row_all, idx_all, val_all, buf0, buf1, stage_v, acc_sh, sem):
    c = lax.axis_index("c")
    s = lax.axis_index("s")

    pltpu.sync_copy(row3.at[s], row_all)
    pltpu.sync_copy(colv.at[s], idx_all)
    pltpu.sync_copy(valv.at[s], val_all)

    def _idx0(i, carry):
        v = idx_all[pl.ds(i * L, L)]
        idx_all[pl.ds(i * L, L)] = v * 4 + c
        return carry
    lax.fori_loop(0, EPS // L, _idx0, 0)

    def gd(b, buf):
        return pltpu.make_async_copy(x4.at[idx_all.at[pl.ds(b * K, K)]], buf, sem)

    def scale_scatter(b, buf):
        def grp(g, carry):
            e0 = b * K + g * L
            for e in range(L):
                vv = plsc.load_gather(val_all, [jnp.full((L,), e0 + e, jnp.int32)])
                r = g * L + e
                for j in range(DH // L):
                    buf[r, pl.ds(j * L, L)] = buf[r, pl.ds(j * L, L)] * vv
            return carry
        lax.fori_loop(0, K // L, grp, 0)
        pltpu.sync_copy(buf, acc_sh.at[row_all.at[b]], add=True)

    for p in range(2):
        if p == 1:
            def _idx1(i, carry):
                idx_all[pl.ds(i * L, L)] = idx_all[pl.ds(i * L, L)] + 2
                return carry
            lax.fori_loop(0, EPS // L, _idx1, 0)

        def _zrow(i, carry):
            for j in range(DH // L):
                stage_v[i, pl.ds(j * L, L)] = jnp.zeros((L,), jnp.float32)
            return carry
        lax.fori_loop(0, STRIPE, _zrow, 0)
        for t in range(ROWS_PER_SUB // STRIPE):
            pltpu.sync_copy(stage_v, acc_sh.at[pl.ds(s * ROWS_PER_SUB + t * STRIPE, STRIPE)])
        plsc.subcore_barrier()

        gd(0, buf0).start()

        def body(j2, carry):
            b0 = j2 * 2
            gd(b0 + 1, buf1).start()
            gd(b0, buf0).wait()
            scale_scatter(b0, buf0)
            gd(b0 + 2, buf0).start()
            gd(b0 + 1, buf1).wait()
            scale_scatter(b0 + 1, buf1)
            return carry
        lax.fori_loop(0, (NBATCH - 1) // 2, body, 0)
        gd(NBATCH - 1, buf0).wait()
        scale_scatter(NBATCH - 1, buf0)

        plsc.subcore_barrier()

        for t in range(ROWS_PER_SUB // STRIPE):
            r0 = s * ROWS_PER_SUB + t * STRIPE
            pltpu.sync_copy(acc_sh.at[pl.ds(r0, STRIPE)], stage_v)
            pltpu.sync_copy(stage_v, out_hbm.at[pl.ds(r0, STRIPE), c + 2 * p])


_gcn = pl.kernel(
    _gcn_body,
    out_type=jax.ShapeDtypeStruct((N, NQ, DH), jnp.float32),
    mesh=plsc.VectorSubcoreMesh(core_axis_name="c", subcore_axis_name="s"),
    compiler_params=pltpu.CompilerParams(
        needs_layout_passes=False, use_tc_tiling_on_sc=False),
    scratch_types=[
        pltpu.VMEM((NBATCH, K), jnp.int32),
        pltpu.VMEM((EPS,), jnp.int32),
        pltpu.VMEM((EPS,), jnp.float32),
        pltpu.VMEM((K, DH), jnp.float32),
        pltpu.VMEM((K, DH), jnp.float32),
        pltpu.VMEM((STRIPE, DH), jnp.float32),
        pltpu.VMEM_SHARED((N, DH), jnp.float32),
        pltpu.SemaphoreType.DMA,
    ],
)


def kernel(edge_index, values, x):
    row3 = edge_index[0].reshape(NS, NBATCH, K)
    colv = edge_index[1].reshape(NS, EPS)
    valv = values.reshape(NS, EPS)
    x4 = x.reshape(NQ * N, DH)
    out4 = _gcn(row3, colv, valv, x4)
    return out4.reshape(N, D)


# speedup vs baseline: 5.2632x; 1.2418x over previous
import jax
import jax.numpy as jnp
from jax import lax
from jax.experimental import pallas as pl
from jax.experimental.pallas import tpu as pltpu
from jax.experimental.pallas import tpu_sc as plsc

N = 10000
E = 160000
D = 256
DH = 64
NQ = 4
NC = 2
NS = 16
L = 16

EPS = E // NS
K = 80
NBATCH = EPS // K
ROWS_PER_SUB = N // NS
STRIPE = 125


def _gcn_body(row3, colv, valv, x4, out_hbm,
              row_all, idx_all, val_all, buf0, buf1, buf2, buf3,
              stage_v, acc_sh, sem, sem_s):
    c = lax.axis_index("c")
    s = lax.axis_index("s")
    bufs = (buf0, buf1, buf2, buf3)

    pltpu.sync_copy(row3.at[s], row_all)
    pltpu.sync_copy(colv.at[s], idx_all)
    pltpu.sync_copy(valv.at[s], val_all)

    def _idx0(i, carry):
        v = idx_all[pl.ds(i * L, L)]
        idx_all[pl.ds(i * L, L)] = v * 4 + c
        return carry
    lax.fori_loop(0, EPS // L, _idx0, 0)

    def gd(b, buf):
        return pltpu.make_async_copy(x4.at[idx_all.at[pl.ds(b * K, K)]], buf, sem)

    def sd_start(b, buf):
        pltpu.async_copy(buf, acc_sh.at[row_all.at[b]], sem_s, add=True)

    def sd_wait(b, buf):
        pltpu.make_async_copy(buf, acc_sh.at[row_all.at[b]], sem_s).wait()

    def scale(b, buf):
        def grp(g, carry):
            e0 = b * K + g * L
            for e in range(L):
                vv = plsc.load_gather(val_all, [jnp.full((L,), e0 + e, jnp.int32)])
                r = g * L + e
                for j in range(DH // L):
                    buf[r, pl.ds(j * L, L)] = buf[r, pl.ds(j * L, L)] * vv
            return carry
        lax.fori_loop(0, K // L, grp, 0)

    for p in range(2):
        if p == 1:
            def _idx1(i, carry):
                idx_all[pl.ds(i * L, L)] = idx_all[pl.ds(i * L, L)] + 2
                return carry
            lax.fori_loop(0, EPS // L, _idx1, 0)

        def _zrow(i, carry):
            for j in range(DH // L):
                stage_v[i, pl.ds(j * L, L)] = jnp.zeros((L,), jnp.float32)
            return carry
        lax.fori_loop(0, STRIPE, _zrow, 0)
        for t in range(ROWS_PER_SUB // STRIPE):
            pltpu.sync_copy(stage_v, acc_sh.at[pl.ds(s * ROWS_PER_SUB + t * STRIPE, STRIPE)])
        plsc.subcore_barrier()

        gd(0, bufs[0]).start()
        gd(1, bufs[1]).start()

        def body(j4, carry):
            b0 = j4 * 4
            for i in range(4):
                b = b0 + i

                @pl.when(b >= 2)
                def _swait():
                    sd_wait(b - 2, bufs[(i + 2) % 4])

                @pl.when(b + 2 <= NBATCH - 1)
                def _gstart():
                    gd(b + 2, bufs[(i + 2) % 4]).start()

                gd(b, bufs[i]).wait()
                scale(b, bufs[i])
                sd_start(b, bufs[i])
            return carry
        lax.fori_loop(0, NBATCH // 4, body, 0)
        bt = NBATCH - 1
        sd_wait(bt - 2, bufs[2])
        gd(bt, bufs[0]).wait()
        scale(bt, bufs[0])
        sd_start(bt, bufs[0])
        sd_wait(NBATCH - 2, bufs[3])
        sd_wait(NBATCH - 1, bufs[0])

        plsc.subcore_barrier()

        for t in range(ROWS_PER_SUB // STRIPE):
            r0 = s * ROWS_PER_SUB + t * STRIPE
            pltpu.sync_copy(acc_sh.at[pl.ds(r0, STRIPE)], stage_v)
            pltpu.sync_copy(stage_v, out_hbm.at[pl.ds(r0, STRIPE), c + 2 * p])


_gcn = pl.kernel(
    _gcn_body,
    out_type=jax.ShapeDtypeStruct((N, NQ, DH), jnp.float32),
    mesh=plsc.VectorSubcoreMesh(core_axis_name="c", subcore_axis_name="s"),
    compiler_params=pltpu.CompilerParams(
        needs_layout_passes=False, use_tc_tiling_on_sc=False),
    scratch_types=[
        pltpu.VMEM((NBATCH, K), jnp.int32),
        pltpu.VMEM((EPS,), jnp.int32),
        pltpu.VMEM((EPS,), jnp.float32),
        pltpu.VMEM((K, DH), jnp.float32),
        pltpu.VMEM((K, DH), jnp.float32),
        pltpu.VMEM((K, DH), jnp.float32),
        pltpu.VMEM((K, DH), jnp.float32),
        pltpu.VMEM((STRIPE, DH), jnp.float32),
        pltpu.VMEM_SHARED((N, DH), jnp.float32),
        pltpu.SemaphoreType.DMA,
        pltpu.SemaphoreType.DMA,
    ],
)


def kernel(edge_index, values, x):
    row3 = edge_index[0].reshape(NS, NBATCH, K)
    colv = edge_index[1].reshape(NS, EPS)
    valv = values.reshape(NS, EPS)
    x4 = x.reshape(NQ * N, DH)
    out4 = _gcn(row3, colv, valv, x4)
    return out4.reshape(N, D)


# speedup vs baseline: 5.3581x; 1.0180x over previous
import jax
import jax.numpy as jnp
from jax import lax
from jax.experimental import pallas as pl
from jax.experimental.pallas import tpu as pltpu
from jax.experimental.pallas import tpu_sc as plsc

N = 10000
E = 160000
D = 256
DH = 64
NQ = 4
NC = 2
NS = 16
L = 16

EPS = E // NS
K = 80
NBATCH = EPS // K
ROWS_PER_SUB = N // NS
STRIPE = 125


def _gcn_body(row3, colv, valv, x4, out_hbm,
              row_all, idx_all, val_all, buf0, buf1, buf2, buf3,
              stage_v, acc_sh, sem, sem_s):
    c = lax.axis_index("c")
    s = lax.axis_index("s")
    bufs = (buf0, buf1, buf2, buf3)

    pltpu.sync_copy(row3.at[s], row_all)
    pltpu.sync_copy(colv.at[s], idx_all)
    pltpu.sync_copy(valv.at[s], val_all)

    def _idx0(i, carry):
        v = idx_all[pl.ds(i * L, L)]
        idx_all[pl.ds(i * L, L)] = v * 4 + c
        return carry
    lax.fori_loop(0, EPS // L, _idx0, 0)

    def gd(b, buf):
        return pltpu.make_async_copy(x4.at[idx_all.at[pl.ds(b * K, K)]], buf, sem)

    def sd_start(b, buf):
        pltpu.async_copy(buf, acc_sh.at[row_all.at[b]], sem_s, add=True)

    def sd_wait(b, buf):
        pltpu.make_async_copy(buf, acc_sh.at[row_all.at[b]], sem_s).wait()

    def scale(b, buf):
        base = b * K
        for g in range(K // L):
            vc = val_all[pl.ds(base + g * L, L)]
            for e in range(L):
                vv = lax.gather(
                    vc, jnp.full((L, 1), e, jnp.int32),
                    lax.GatherDimensionNumbers(
                        offset_dims=(), collapsed_slice_dims=(0,),
                        start_index_map=(0,)),
                    slice_sizes=(1,),
                    mode=lax.GatherScatterMode.PROMISE_IN_BOUNDS)
                r = g * L + e
                for j in range(DH // L):
                    buf[r, pl.ds(j * L, L)] = buf[r, pl.ds(j * L, L)] * vv

    for p in range(2):
        if p == 1:
            def _idx1(i, carry):
                idx_all[pl.ds(i * L, L)] = idx_all[pl.ds(i * L, L)] + 2
                return carry
            lax.fori_loop(0, EPS // L, _idx1, 0)

        def _zrow(i, carry):
            for j in range(DH // L):
                stage_v[i, pl.ds(j * L, L)] = jnp.zeros((L,), jnp.float32)
            return carry
        lax.fori_loop(0, STRIPE, _zrow, 0)
        for t in range(ROWS_PER_SUB // STRIPE):
            pltpu.sync_copy(stage_v, acc_sh.at[pl.ds(s * ROWS_PER_SUB + t * STRIPE, STRIPE)])
        plsc.subcore_barrier()

        gd(0, bufs[0]).start()
        gd(1, bufs[1]).start()

        def body(j4, carry):
            b0 = j4 * 4
            for i in range(4):
                b = b0 + i

                @pl.when(b >= 2)
                def _swait():
                    sd_wait(b - 2, bufs[(i + 2) % 4])

                @pl.when(b + 2 <= NBATCH - 1)
                def _gstart():
                    gd(b + 2, bufs[(i + 2) % 4]).start()

                gd(b, bufs[i]).wait()
                scale(b, bufs[i])
                sd_start(b, bufs[i])
            return carry
        lax.fori_loop(0, NBATCH // 4, body, 0)
        bt = NBATCH - 1
        sd_wait(bt - 2, bufs[2])
        gd(bt, bufs[0]).wait()
        scale(bt, bufs[0])
        sd_start(bt, bufs[0])
        sd_wait(NBATCH - 2, bufs[3])
        sd_wait(NBATCH - 1, bufs[0])

        plsc.subcore_barrier()

        for t in range(ROWS_PER_SUB // STRIPE):
            r0 = s * ROWS_PER_SUB + t * STRIPE
            pltpu.sync_copy(acc_sh.at[pl.ds(r0, STRIPE)], stage_v)
            pltpu.sync_copy(stage_v, out_hbm.at[pl.ds(r0, STRIPE), c + 2 * p])


_gcn = pl.kernel(
    _gcn_body,
    out_type=jax.ShapeDtypeStruct((N, NQ, DH), jnp.float32),
    mesh=plsc.VectorSubcoreMesh(core_axis_name="c", subcore_axis_name="s"),
    compiler_params=pltpu.CompilerParams(
        needs_layout_passes=False, use_tc_tiling_on_sc=False),
    scratch_types=[
        pltpu.VMEM((NBATCH, K), jnp.int32),
        pltpu.VMEM((EPS,), jnp.int32),
        pltpu.VMEM((EPS,), jnp.float32),
        pltpu.VMEM((K, DH), jnp.float32),
        pltpu.VMEM((K, DH), jnp.float32),
        pltpu.VMEM((K, DH), jnp.float32),
        pltpu.VMEM((K, DH), jnp.float32),
        pltpu.VMEM((STRIPE, DH), jnp.float32),
        pltpu.VMEM_SHARED((N, DH), jnp.float32),
        pltpu.SemaphoreType.DMA,
        pltpu.SemaphoreType.DMA,
    ],
)


def kernel(edge_index, values, x):
    row3 = edge_index[0].reshape(NS, NBATCH, K)
    colv = edge_index[1].reshape(NS, EPS)
    valv = values.reshape(NS, EPS)
    x4 = x.reshape(NQ * N, DH)
    out4 = _gcn(row3, colv, valv, x4)
    return out4.reshape(N, D)
